# baseline (speedup 1.0000x reference)
import jax
import jax.numpy as jnp
from jax.experimental import pallas as pl

_K = 256
_CHUNK = 2048


def _matmul_body(img_ref, text_ref, out_ref):
    i = pl.program_id(0)
    logits = jax.lax.dot_general(
        img_ref[...], text_ref[...],
        dimension_numbers=(((1,), (1,)), ((), ())),
        preferred_element_type=jnp.float32,
    )
    n_total = 100000
    lane = i * _CHUNK + jax.lax.broadcasted_iota(jnp.int32, logits.shape, 1)
    out_ref[...] = jnp.where(lane < n_total, logits, -jnp.inf)


def kernel(img_feat, text_feats):
    n, d = text_feats.shape
    m = img_feat.shape[0]
    n_pad = ((n + _CHUNK - 1) // _CHUNK) * _CHUNK
    grid = n_pad // _CHUNK
    logits = pl.pallas_call(
        _matmul_body,
        grid=(grid,),
        in_specs=[
            pl.BlockSpec((m, d), lambda i: (0, 0)),
            pl.BlockSpec((_CHUNK, d), lambda i: (i, 0)),
        ],
        out_specs=pl.BlockSpec((m, _CHUNK), lambda i: (0, i)),
        out_shape=jax.ShapeDtypeStruct((m, n_pad), jnp.float32),
    )(img_feat, text_feats)
    _, idx = jax.lax.top_k(logits, _K)
    return idx


# speedup vs baseline: 3.0005x; 3.0005x over previous
import jax
import jax.numpy as jnp
from jax import lax
from jax.experimental import pallas as pl
from jax.experimental.pallas import tpu as pltpu
from jax.experimental.pallas import tpu_sc as plsc

_K = 256
_CHUNK = 2048
_N = 100000
_N_PAD = 100352
_ROWS = 8
_SH = _N_PAD // 4
_NV = _SH // 16
_M = 4 * _K
_MV = _M // 16
_SH_PAD = 64


def _matmul_body(img_ref, text_ref, out_ref):
    i = pl.program_id(0)
    logits = lax.dot_general(
        img_ref[...], text_ref[...],
        dimension_numbers=(((1,), (1,)), ((), ())),
        preferred_element_type=jnp.float32,
    )
    gl = i * _CHUNK + lax.broadcasted_iota(jnp.int32, logits.shape, 1)
    out_ref[...] = jnp.where(gl < _N, logits, -jnp.inf)


def _compute_logits(img_feat, text_feats):
    m, d = img_feat.shape
    return pl.pallas_call(
        _matmul_body,
        grid=(_N_PAD // _CHUNK,),
        in_specs=[
            pl.BlockSpec((m, d), lambda i: (0, 0)),
            pl.BlockSpec((_CHUNK, d), lambda i: (i, 0)),
        ],
        out_specs=pl.BlockSpec((m, _CHUNK), lambda i: (0, i)),
        out_shape=jax.ShapeDtypeStruct((m, _N_PAD), jnp.float32),
    )(img_feat, text_feats)


def _lane():
    return lax.iota(jnp.int32, 16)


def _splat(x):
    return jnp.full((16,), 0, jnp.int32) + x


def _sc_body(log_hbm, out_hbm, vals, ukey, cbuf_k, cbuf_i, hist, rhist,
             cand_k, cand_i, eq_i, pub_k, pub_i, mkey, midx, meq, orow, sh):
    c = lax.axis_index("c")
    s = lax.axis_index("s")
    row_local = s // 4
    shard = s % 4
    row = c * 4 + row_local
    base = row * _N_PAD + shard * _SH
    ridx0 = shard * _SH
    lane = _lane()
    ones = jnp.full((16,), 1, jnp.int32)
    zeros = jnp.full((16,), 0, jnp.int32)

    pltpu.sync_copy(log_hbm.at[pl.ds(base, _SH)], vals)

    def zero_hist(i, carry):
        hist[pl.ds(i * 16, 16)] = zeros
        return carry

    def hist_add(binv, mask=None):
        addr = binv * 16 + lane
        cur = plsc.load_gather(hist, [addr])
        if mask is None:
            plsc.store_scatter(hist, [addr], cur + ones)
        else:
            plsc.store_scatter(hist, [addr], cur + ones, mask=mask)

    def reduce_hist(j, carry):
        acc = zeros
        for l in range(16):
            acc = acc + plsc.load_gather(hist, [(j * 16 + lane) * 16 + l])
        rhist[pl.ds(j * 16, 16)] = acc
        return carry

    def pick_byte(k_rem):
        found = jnp.int32(0)
        bstar = jnp.int32(0)
        carry_cnt = jnp.int32(0)
        for v in range(15, -1, -1):
            rh = rhist[pl.ds(v * 16, 16)]
            cum = plsc.cumsum(lax.rev(rh, (0,))) + carry_cnt
            ge = cum >= k_rem
            cnt = jnp.sum(ge.astype(jnp.int32))
            jsc = jnp.max(plsc.all_reduce_ffs(ge))
            hit = jnp.logical_and(found == 0, cnt > 0)
            bstar = jnp.where(hit, jnp.int32(v * 16 + 15) - jsc, bstar)
            found = jnp.where(hit, jnp.int32(1), found)
            carry_cnt = carry_cnt + jnp.sum(rh)
        acc = zeros
        for v in range(16):
            binids = v * 16 + lane
            rh = rhist[pl.ds(v * 16, 16)]
            acc = acc + jnp.where(binids > bstar, rh, 0)
        s_excl = jnp.sum(acc)
        return bstar, s_excl

    lax.fori_loop(0, 256, zero_hist, 0)

    def p0(i, carry):
        v = vals[pl.ds(i * 16, 16)]
        b = lax.bitcast_convert_type(v, jnp.int32)
        u = lax.bitcast_convert_type(
            jnp.where(b >= 0, b ^ jnp.int32(-2147483648), ~b), jnp.uint32)
        ukey[pl.ds(i * 16, 16)] = lax.bitcast_convert_type(u, jnp.int32)
        hist_add((u >> jnp.uint32(24)).astype(jnp.int32))
        return carry
    lax.fori_loop(0, _NV, p0, 0, unroll=8)

    lax.fori_loop(0, 16, reduce_hist, 0)
    b1, s_excl = pick_byte(jnp.int32(_K))
    k_rem = jnp.int32(_K) - s_excl

    lax.fori_loop(0, 256, zero_hist, 0)

    def p1(i, carry):
        cg, ptr = carry
        u = lax.bitcast_convert_type(ukey[pl.ds(i * 16, 16)], jnp.uint32)
        ui = lax.bitcast_convert_type(u, jnp.int32)
        gidx = ridx0 + i * 16 + lane
        b3 = (u >> jnp.uint32(24)).astype(jnp.int32)
        mg = b3 > b1
        posg = cg + plsc.cumsum(mg.astype(jnp.int32)) - 1
        sm = mg & (posg < _K)
        plsc.store_scatter(cand_k, [posg], ui, mask=sm)
        plsc.store_scatter(cand_i, [posg], gidx, mask=sm)
        cg = cg + plsc.all_reduce_population_count(mg)
        me = b3 == b1
        pose = ptr + plsc.cumsum(me.astype(jnp.int32)) - 1
        plsc.store_scatter(cbuf_k, [pose], ui, mask=me)
        plsc.store_scatter(cbuf_i, [pose], ridx0 + i * 16 + lane, mask=me)
        ptr = ptr + plsc.all_reduce_population_count(me)
        hist_add(((u >> jnp.uint32(16)) & jnp.uint32(0xFF)).astype(jnp.int32),
                 mask=me)
        return cg, ptr
    cg, ptr = lax.fori_loop(0, _NV, p1, (zeros, zeros), unroll=4)
    lax.fori_loop(0, 16, reduce_hist, 0)
    b2, s_excl = pick_byte(k_rem)
    k_rem = k_rem - s_excl
    nset = jnp.max(ptr)

    def eq_round(bcur, shift, cg, nset, cap):
        lax.fori_loop(0, 256, zero_hist, 0)

        def pb(i, carry):
            cgi, wp = carry
            u = lax.bitcast_convert_type(cbuf_k[pl.ds(i * 16, 16)],
                                         jnp.uint32)
            ui = lax.bitcast_convert_type(u, jnp.int32)
            gi = cbuf_i[pl.ds(i * 16, 16)]
            valid = (i * 16 + lane) < nset
            fld = ((u >> jnp.uint32(shift)) & jnp.uint32(0xFF)).astype(
                jnp.int32)
            mg = valid & (fld > bcur)
            posg = cgi + plsc.cumsum(mg.astype(jnp.int32)) - 1
            sm = mg & (posg < cap)
            plsc.store_scatter(cand_k, [posg], ui, mask=sm)
            plsc.store_scatter(cand_i, [posg], gi, mask=sm)
            cgi = cgi + plsc.all_reduce_population_count(mg)
            me = valid & (fld == bcur)
            pose = wp + plsc.cumsum(me.astype(jnp.int32)) - 1
            plsc.store_scatter(cbuf_k, [pose], ui, mask=me)
            plsc.store_scatter(cbuf_i, [pose], gi, mask=me)
            wp = wp + plsc.all_reduce_population_count(me)
            hist_add(((u >> jnp.uint32(shift - 8)) &
                      jnp.uint32(0xFF)).astype(jnp.int32), mask=me)
            return cgi, wp
        nv = (nset + 15) // 16
        cg2, wp = lax.fori_loop(0, nv, pb, (cg, zeros))
        lax.fori_loop(0, 16, reduce_hist, 0)
        return cg2, jnp.max(wp)

    cg, nset = eq_round(b2, 16, cg, nset, _K)
    b3_, s_excl = pick_byte(k_rem)
    k_rem = k_rem - s_excl

    cg, nset = eq_round(b3_, 8, cg, nset, _K)
    b4, s_excl = pick_byte(k_rem)
    k_rem = k_rem - s_excl

    def p4(i, carry):
        cgi, ce = carry
        u = lax.bitcast_convert_type(cbuf_k[pl.ds(i * 16, 16)], jnp.uint32)
        ui = lax.bitcast_convert_type(u, jnp.int32)
        gi = cbuf_i[pl.ds(i * 16, 16)]
        valid = (i * 16 + lane) < nset
        b0 = (u & jnp.uint32(0xFF)).astype(jnp.int32)
        mg = valid & (b0 > b4)
        posg = cgi + plsc.cumsum(mg.astype(jnp.int32)) - 1
        sm = mg & (posg < _K)
        plsc.store_scatter(cand_k, [posg], ui, mask=sm)
        plsc.store_scatter(cand_i, [posg], gi, mask=sm)
        cgi = cgi + plsc.all_reduce_population_count(mg)
        me = valid & (b0 == b4)
        pose = ce + plsc.cumsum(me.astype(jnp.int32)) - 1
        me = me & (pose < _K)
        plsc.store_scatter(eq_i, [pose], gi, mask=me)
        ce = ce + plsc.all_reduce_population_count(me)
        return cgi, ce
    nv4 = (nset + 15) // 16
    cg, _ce = lax.fori_loop(0, nv4, p4, (cg, zeros))
    cgs = jnp.max(cg)

    tkey = (lax.shift_left(b1, 24) | lax.shift_left(b2, 16)
            | lax.shift_left(b3_, 8) | b4)

    for ci in range(16):
        posv = ci * 16 + lane
        in_g = posv < cgs
        toff = jnp.maximum(posv - cgs, 0)
        toff = jnp.minimum(toff, _K - 1)
        tie_i = plsc.load_gather(eq_i, [toff])
        pub_k[pl.ds(ci * 16, 16)] = jnp.where(
            in_g, cand_k[pl.ds(ci * 16, 16)], _splat(tkey))
        pub_i[pl.ds(ci * 16, 16)] = jnp.where(
            in_g, cand_i[pl.ds(ci * 16, 16)], tie_i)

    for S in range(16):
        @pl.when(s == S)
        def _(S=S):
            pltpu.sync_copy(pub_k, sh.at[pl.ds(_SH_PAD + S * 512, _K)])
            pltpu.sync_copy(pub_i, sh.at[pl.ds(_SH_PAD + S * 512 + _K, _K)])
    plsc.subcore_barrier()

    for S in (0, 4, 8, 12):
        @pl.when(s == S)
        def _(S=S):
            for j in range(4):
                off = _SH_PAD + (S + j) * 512
                pltpu.sync_copy(sh.at[pl.ds(off, _K)],
                                mkey.at[pl.ds(j * _K, _K)])
                pltpu.sync_copy(sh.at[pl.ds(off + _K, _K)],
                                midx.at[pl.ds(j * _K, _K)])

    @pl.when(shard == 0)
    def _():
        def seed(i, carry):
            cbuf_k[pl.ds(i * 16, 16)] = mkey[pl.ds(i * 16, 16)]
            cbuf_i[pl.ds(i * 16, 16)] = midx[pl.ds(i * 16, 16)]
            return carry
        lax.fori_loop(0, _MV, seed, 0)

        lax.fori_loop(0, 256, zero_hist, 0)

        def m0(i, carry):
            u = lax.bitcast_convert_type(mkey[pl.ds(i * 16, 16)], jnp.uint32)
            hist_add((u >> jnp.uint32(24)).astype(jnp.int32))
            return carry
        lax.fori_loop(0, _MV, m0, 0)
        lax.fori_loop(0, 16, reduce_hist, 0)
        B1, sx = pick_byte(jnp.int32(_K))
        krem = jnp.int32(_K) - sx

        mcg, mns = eq_round(B1, 24, zeros, jnp.int32(_M), _K)
        B2, sx = pick_byte(krem)
        krem = krem - sx

        mcg, mns = eq_round(B2, 16, mcg, mns, _K)
        B3, sx = pick_byte(krem)
        krem = krem - sx

        mcg, mns = eq_round(B3, 8, mcg, mns, _K)
        B4, sx = pick_byte(krem)
        krem = krem - sx

        def m4(i, carry):
            cgi, ce = carry
            u = lax.bitcast_convert_type(cbuf_k[pl.ds(i * 16, 16)],
                                         jnp.uint32)
            ui = lax.bitcast_convert_type(u, jnp.int32)
            gi = cbuf_i[pl.ds(i * 16, 16)]
            valid = (i * 16 + lane) < mns
            b0 = (u & jnp.uint32(0xFF)).astype(jnp.int32)
            mg = valid & (b0 > B4)
            posg = cgi + plsc.cumsum(mg.astype(jnp.int32)) - 1
            sm = mg & (posg < _K)
            plsc.store_scatter(cand_k, [posg], ui, mask=sm)
            plsc.store_scatter(cand_i, [posg], gi, mask=sm)
            cgi = cgi + plsc.all_reduce_population_count(mg)
            me = valid & (b0 == B4)
            pose = ce + plsc.cumsum(me.astype(jnp.int32)) - 1
            me = me & (pose < _M)
            plsc.store_scatter(meq, [pose], gi, mask=me)
            ce = ce + plsc.all_reduce_population_count(me)
            return cgi, ce
        nvm = (mns + 15) // 16
        mcg, mce = lax.fori_loop(0, nvm, m4, (mcg, zeros))
        g_cnt = jnp.max(mcg)
        e_cnt = jnp.max(mce)

        def rj(j, ranks):
            kj = lax.bitcast_convert_type(
                plsc.load_gather(cand_k, [_splat(j)]), jnp.uint32)
            ij = plsc.load_gather(cand_i, [_splat(j)])
            out = []
            for ci in range(16):
                posv = ci * 16 + lane
                ku = lax.bitcast_convert_type(cand_k[pl.ds(ci * 16, 16)],
                                              jnp.uint32)
                iu = cand_i[pl.ds(ci * 16, 16)]
                gt = (kj > ku) | ((kj == ku) & (ij < iu))
                gt = gt & (posv < g_cnt)
                out.append(ranks[ci] + gt.astype(jnp.int32))
            return tuple(out)
        ranks0 = tuple([zeros] * 16)
        ranks = lax.fori_loop(0, g_cnt, rj, ranks0)

        for ci in range(16):
            posv = ci * 16 + lane
            mi = cand_i[pl.ds(ci * 16, 16)]
            plsc.store_scatter(orow, [ranks[ci]], mi,
                               mask=(posv < g_cnt) & (ranks[ci] < _K))

        def tj(j, carry):
            ij = jnp.max(plsc.load_gather(meq, [_splat(j)]))

            def cnt_lt(i, acc):
                v = meq[pl.ds(i * 16, 16)]
                m = ((i * 16 + lane) < e_cnt) & (v < ij)
                return acc + plsc.all_reduce_population_count(m)
            nve = (e_cnt + 15) // 16
            r = jnp.max(lax.fori_loop(0, nve, cnt_lt, zeros))
            pos = g_cnt + r
            ok = (r < krem) & (pos < _K)
            plsc.store_scatter(orow, [_splat(pos)], _splat(ij),
                               mask=(lane == 0) & ok)
            return carry
        lax.fori_loop(0, e_cnt, tj, 0)

        pltpu.sync_copy(orow, out_hbm.at[pl.ds(row * _K, _K)])


def _sc_topk(logits_flat):
    mesh = plsc.VectorSubcoreMesh(core_axis_name="c", subcore_axis_name="s")
    kfn = pl.kernel(
        _sc_body,
        out_type=jax.ShapeDtypeStruct((_ROWS * _K,), jnp.int32),
        mesh=mesh,
        compiler_params=pltpu.CompilerParams(needs_layout_passes=False),
        scratch_types=[
            pltpu.VMEM((_SH,), jnp.float32),
            pltpu.VMEM((_SH,), jnp.int32),
            pltpu.VMEM((_SH,), jnp.int32),
            pltpu.VMEM((_SH,), jnp.int32),
            pltpu.VMEM((4096,), jnp.int32),
            pltpu.VMEM((256,), jnp.int32),
            pltpu.VMEM((_K,), jnp.int32),
            pltpu.VMEM((_K,), jnp.int32),
            pltpu.VMEM((_K,), jnp.int32),
            pltpu.VMEM((_K,), jnp.int32),
            pltpu.VMEM((_K,), jnp.int32),
            pltpu.VMEM((_M,), jnp.int32),
            pltpu.VMEM((_M,), jnp.int32),
            pltpu.VMEM((_M,), jnp.int32),
            pltpu.VMEM((_K,), jnp.int32),
            pltpu.VMEM_SHARED((_SH_PAD + 16 * 512,), jnp.int32),
        ],
    )
    return kfn(logits_flat)


def kernel(img_feat, text_feats):
    logits = _compute_logits(img_feat, text_feats)
    idx = _sc_topk(logits.reshape(-1))
    return idx.reshape(_ROWS, _K)
